# baseline (speedup 1.0000x reference)
import functools

import jax
import jax.numpy as jnp
from jax import lax
from jax.experimental import pallas as pl
from jax.experimental.pallas import tpu as pltpu
from jax.experimental.pallas import tpu_sc as plsc

N = 10000
E = 320000
D = 128
R = 7
L = 4
G8 = 8

NP = 10240
BN = 512
NC = 2
NS = 16
NT = NC * NS
EPT = NP
EP = NT * EPT
CHUNK = 4096
NCH = 18
NSLOT = NCH * CHUNK
CAP = 10240
BLK = 128
ZR = 64
NBLK = NCH * NT * (CAP // BLK)

_mesh = plsc.VectorSubcoreMesh(core_axis_name="c", subcore_axis_name="s")


def _sqrt_sc(sq):
    i = plsc.bitcast(sq, jnp.int32)
    y = plsc.bitcast((i >> 1) + 0x1FBD1DF6, jnp.float32)
    for _ in range(3):
        y = 0.5 * (y + sq / y)
    return y


def _lane(vec, lane):
    m = lax.broadcasted_iota(jnp.int32, (16,), 0) == lane
    return jnp.sum(jnp.where(m, vec, 0))


def _splat(val):
    return jnp.full((16,), val, jnp.int32)


_LANE0 = None


def _pre_body(posf, srcg, dstg, relg,
              scl_out, ilist_out, counts_out,
              pos_v, srcb, dstb, relb,
              seqb, distb, onesb, i0b, i1b, i2b,
              stage_s, stage_l, cntv, zbuf, scl_sh, sem, semw):
    cid = lax.axis_index("c")
    sid = lax.axis_index("s")
    t = cid * NS + sid

    pltpu.sync_copy(posf, pos_v)
    pltpu.sync_copy(srcg.at[pl.ds(t * EPT, EPT)], srcb)
    pltpu.sync_copy(dstg.at[pl.ds(t * EPT, EPT)], dstb)
    pltpu.sync_copy(relg.at[pl.ds(t * EPT, EPT)], relb)

    ones16 = jnp.ones((16,), jnp.float32)
    for kk in range(8):
        onesb[pl.ds(kk * 16, 16)] = ones16

    z16 = jnp.zeros((16,), jnp.float32)
    def zfill(i, _):
        zbuf[pl.ds(i * 16, 16)] = z16
        return 0
    lax.fori_loop(0, 2048 // 16, zfill, 0)
    myz = NSLOT * 8 // NS
    def zloop(i, _):
        pltpu.sync_copy(zbuf, scl_sh.at[pl.ds(sid * myz + i * 2048, 2048)])
        return 0
    lax.fori_loop(0, myz // 2048, zloop, 0)
    plsc.subcore_barrier()

    def p1(j, _):
        base = j * BLK
        for kk in range(BLK // 16):
            o = base + kk * 16
            lo = kk * 16
            s = srcb[pl.ds(o, 16)]
            d = dstb[pl.ds(o, 16)]
            r = relb[pl.ds(o, 16)]
            slot = r * NP + d
            relb[pl.ds(o, 16)] = slot
            e8 = slot * 8
            i0b[pl.ds(lo, 16)] = e8
            i1b[pl.ds(lo, 16)] = e8 + 1
            i2b[pl.ds(lo, 16)] = e8 + 2
            seq = jnp.abs(s - d).astype(jnp.float32)
            px = plsc.load_gather(pos_v, [s])
            py = plsc.load_gather(pos_v, [s + NP])
            pz = plsc.load_gather(pos_v, [s + 2 * NP])
            qx = plsc.load_gather(pos_v, [d])
            qy = plsc.load_gather(pos_v, [d + NP])
            qz = plsc.load_gather(pos_v, [d + 2 * NP])
            dx = px - qx + 1e-6
            dy = py - qy + 1e-6
            dz = pz - qz + 1e-6
            seqb[pl.ds(lo, 16)] = seq
            distb[pl.ds(lo, 16)] = _sqrt_sc(dx * dx + dy * dy + dz * dz)
        pltpu.sync_copy(onesb, scl_sh.at[i0b], add=True)
        pltpu.sync_copy(seqb, scl_sh.at[i1b], add=True)
        pltpu.sync_copy(distb, scl_sh.at[i2b], add=True)
        return 0
    lax.fori_loop(0, EPT // BLK, p1, 0)

    lane0 = lax.broadcasted_iota(jnp.int32, (16,), 0) == 0
    pad_lsl16 = jnp.full((16,), CHUNK, jnp.int32)
    zero16 = jnp.zeros((16,), jnp.int32)
    for c in range(NCH):
        def cstep(i, cur):
            o = i * 16
            sl = relb[pl.ds(o, 16)]
            m = (sl >> 12) == c
            plsc.store_compressed(stage_s.at[pl.ds(cur, 16)],
                                  srcb[pl.ds(o, 16)], mask=m)
            plsc.store_compressed(stage_l.at[pl.ds(cur, 16)],
                                  sl & (CHUNK - 1), mask=m)
            pc = plsc.all_reduce_population_count(m)
            return cur + _lane(pc, 0)
        cursor = lax.fori_loop(0, EPT // 16, cstep, 0)
        target = ((cursor + BLK - 1) // BLK) * BLK
        def padstep(i, _):
            stage_s[pl.ds(cursor + i * 16, 16)] = zero16
            stage_l[pl.ds(cursor + i * 16, 16)] = pad_lsl16
            return 0
        lax.fori_loop(0, (target - cursor + 15) // 16, padstep, 0)
        nb = target // BLK
        xbase = (c * NT + t) * (CAP // BLK)
        def wout(jb, _):
            pltpu.async_copy(stage_s.at[pl.ds(jb * BLK, BLK)],
                             ilist_out.at[xbase + jb, 0], semw)
            pltpu.async_copy(stage_l.at[pl.ds(jb * BLK, BLK)],
                             ilist_out.at[xbase + jb, 1], semw)
            return 0
        lax.fori_loop(0, nb, wout, 0)
        def wdrain(jb, _):
            pltpu.make_async_copy(stage_s.at[pl.ds(0, BLK)],
                                  ilist_out.at[xbase, 0], semw).wait()
            pltpu.make_async_copy(stage_s.at[pl.ds(0, BLK)],
                                  ilist_out.at[xbase, 1], semw).wait()
            return 0
        lax.fori_loop(0, nb, wdrain, 0)
        plsc.store_scatter(cntv, [_splat(c)], jnp.full((16,), nb, jnp.int32),
                           mask=lane0)

    pltpu.sync_copy(cntv, counts_out.at[pl.ds(t * 32, 32)])

    plsc.subcore_barrier()
    pltpu.sync_copy(scl_sh.at[pl.ds(sid * myz, myz)],
                    scl_out.at[pl.ds(cid * (NSLOT * 8) + sid * myz, myz)])


@functools.partial(
    pl.kernel,
    out_type=[
        jax.ShapeDtypeStruct((NC * NSLOT * 8,), jnp.float32),
        jax.ShapeDtypeStruct((NBLK, 2, BLK), jnp.int32),
        jax.ShapeDtypeStruct((NT * 32,), jnp.int32),
    ],
    mesh=_mesh,
    compiler_params=pltpu.CompilerParams(needs_layout_passes=False),
    scratch_types=[
        pltpu.VMEM((3 * NP,), jnp.float32),
        pltpu.VMEM((EPT,), jnp.int32),
        pltpu.VMEM((EPT,), jnp.int32),
        pltpu.VMEM((EPT,), jnp.int32),
        pltpu.VMEM((BLK,), jnp.float32),
        pltpu.VMEM((BLK,), jnp.float32),
        pltpu.VMEM((BLK,), jnp.float32),
        pltpu.VMEM((BLK,), jnp.int32),
        pltpu.VMEM((BLK,), jnp.int32),
        pltpu.VMEM((BLK,), jnp.int32),
        pltpu.VMEM((CAP + 32,), jnp.int32),
        pltpu.VMEM((CAP + 32,), jnp.int32),
        pltpu.VMEM((32,), jnp.int32),
        pltpu.VMEM((2048,), jnp.float32),
        pltpu.VMEM_SHARED((NSLOT * 8,), jnp.float32),
        pltpu.SemaphoreType.DMA,
        pltpu.SemaphoreType.DMA,
    ],
)
def _preprocess(posf, srcg, dstg, relg, scl_out, ilist_out,
                counts_out, *scratch):
    _pre_body(posf, srcg, dstg, relg, scl_out, ilist_out,
              counts_out, *scratch)


def _scat_body(h_hbm, ilist, counts, g_out,
               counts_v, idx2, rowbuf, zv, gbuf,
               si0, si1, si2, sg0, sg1, sg2, ss0, ss1, ss2):
    cid = lax.axis_index("c")
    sid = lax.axis_index("s")
    sems_i = (si0, si1, si2)
    sems_g = (sg0, sg1, sg2)
    sems_s = (ss0, ss1, ss2)

    pltpu.sync_copy(counts, counts_v)
    z16 = jnp.zeros((16,), jnp.float32)
    def zfill(i, _):
        zv[i, pl.ds(0, 16)] = z16
        for kk in range(1, D // 16):
            zv[i, pl.ds(kk * 16, 16)] = z16
        return 0
    lax.fori_loop(0, ZR, zfill, 0)

    rows_per_tile = CHUNK // NS
    for k in range(NCH // NC):
        c = NC * k + cid
        for zz in range(rows_per_tile // ZR):
            pltpu.sync_copy(zv, gbuf.at[pl.ds(sid * rows_per_tile + zz * ZR, ZR)])
        plsc.subcore_barrier()

        t0 = sid
        t1 = sid + NS
        nb0 = _lane(plsc.load_gather(counts_v, [_splat(t0 * 32 + c)]), 0)
        nb1 = _lane(plsc.load_gather(counts_v, [_splat(t1 * 32 + c)]), 0)
        nbt = nb0 + nb1
        x0 = (c * NT + t0) * (CAP // BLK)
        x1 = (c * NT + t1) * (CAP // BLK)

        def body(j, _):
            xb = jnp.where(j < nb0, x0 + j, x1 + (j - nb0))
            pltpu.sync_copy(ilist.at[xb], idx2.at[0])
            pltpu.async_copy(h_hbm.at[idx2.at[0, 0]], rowbuf.at[0],
                             sems_g[0]).wait()
            pltpu.sync_copy(rowbuf.at[0], gbuf.at[idx2.at[0, 1]], add=True)
            return 0
        lax.fori_loop(0, nbt, body, 0)

        plsc.subcore_barrier()
        pltpu.sync_copy(
            gbuf.at[pl.ds(sid * rows_per_tile, rows_per_tile)],
            g_out.at[pl.ds(c * CHUNK + sid * rows_per_tile, rows_per_tile)])


@functools.partial(
    pl.kernel,
    out_type=jax.ShapeDtypeStruct((NSLOT, D), jnp.float32),
    mesh=_mesh,
    compiler_params=pltpu.CompilerParams(needs_layout_passes=False),
    scratch_types=[
        pltpu.VMEM((32 * NT,), jnp.int32),
        pltpu.VMEM((3, 2, BLK), jnp.int32),
        pltpu.VMEM((3, BLK, D), jnp.float32),
        pltpu.VMEM((ZR, D), jnp.float32),
        pltpu.VMEM_SHARED((CHUNK + 1, D), jnp.float32),
        pltpu.SemaphoreType.DMA,
        pltpu.SemaphoreType.DMA,
        pltpu.SemaphoreType.DMA,
        pltpu.SemaphoreType.DMA,
        pltpu.SemaphoreType.DMA,
        pltpu.SemaphoreType.DMA,
        pltpu.SemaphoreType.DMA,
        pltpu.SemaphoreType.DMA,
        pltpu.SemaphoreType.DMA,
    ],
)
def _scatter(h_hbm, ilist, counts, g_out, *scratch):
    _scat_body(h_hbm, ilist, counts, g_out, *scratch)


def _dense_layer_body(g_ref, sa_ref, sb_ref, b_ref, c_ref, uv_ref, li_ref,
                      an_ref, wl_ref, ws_ref, h_ref, hn_ref):
    li = li_ref[...]
    acc = jnp.dot(li, ws_ref[...], preferred_element_type=jnp.float32)
    bmat = b_ref[...]
    for r in range(R):
        sa = sa_ref[r]
        sb = sb_ref[r]
        cnt = sa[:, 0:1] + sb[:, 0:1]
        s1 = sa[:, 1:2] + sb[:, 1:2]
        s2 = sa[:, 2:3] + sb[:, 2:3]
        upd = (g_ref[r] + cnt * (bmat + c_ref[r:r + 1, :])
               + s1 * uv_ref[0:1, :] + s2 * uv_ref[1:2, :])
        acc += jnp.dot(upd, wl_ref[r], preferred_element_type=jnp.float32)
    h = jnp.maximum(acc + uv_ref[2:3, :], 0.0) + li
    h_ref[...] = h
    hn_ref[...] = h + an_ref[...]


def _dense_layer(g3, sa, sbp, bmat, cmat, uv, li, an, wl3, ws):
    nb = NP // BN
    return pl.pallas_call(
        _dense_layer_body,
        grid=(nb,),
        in_specs=[
            pl.BlockSpec((R, BN, D), lambda i: (0, i, 0)),
            pl.BlockSpec((R, BN, 8), lambda i: (0, i, 0)),
            pl.BlockSpec((R, BN, 8), lambda i: (0, i, 0)),
            pl.BlockSpec((BN, D), lambda i: (i, 0)),
            pl.BlockSpec((R, D), lambda i: (0, 0)),
            pl.BlockSpec((8, D), lambda i: (0, 0)),
            pl.BlockSpec((BN, D), lambda i: (i, 0)),
            pl.BlockSpec((BN, D), lambda i: (i, 0)),
            pl.BlockSpec((R, D, D), lambda i: (0, 0, 0)),
            pl.BlockSpec((D, D), lambda i: (0, 0)),
        ],
        out_specs=[pl.BlockSpec((BN, D), lambda i: (i, 0)),
                   pl.BlockSpec((BN, D), lambda i: (i, 0))],
        out_shape=[jax.ShapeDtypeStruct((NP, D), jnp.float32)] * 2,
    )(g3, sa, sbp, bmat, cmat, uv, li, an, wl3, ws)


def _precompute_body(x_ref, w_ref, ab_ref, h0_ref):
    xb = x_ref[...]
    ab = jnp.dot(xb, w_ref[...], preferred_element_type=jnp.float32)
    ab_ref[...] = ab
    h0_ref[...] = xb + ab[:, :D]


def _precompute(x, wcat):
    return pl.pallas_call(
        _precompute_body,
        grid=(NP // BN,),
        in_specs=[pl.BlockSpec((BN, D), lambda i: (i, 0)),
                  pl.BlockSpec((D, 2 * L * D), lambda i: (0, 0))],
        out_specs=[pl.BlockSpec((BN, 2 * L * D), lambda i: (i, 0)),
                   pl.BlockSpec((BN, D), lambda i: (i, 0))],
        out_shape=[jax.ShapeDtypeStruct((NP, 2 * L * D), jnp.float32),
                   jax.ShapeDtypeStruct((NP, D), jnp.float32)],
    )(x, wcat)


def _pool_body(b_ref, xf_ref, out_ref):
    i = pl.program_id(0)
    oh = (b_ref[...] == jax.lax.broadcasted_iota(jnp.int32, (1, G8), 1))
    part = jax.lax.dot_general(oh.astype(jnp.float32), xf_ref[...],
                               (((0,), (0,)), ((), ())),
                               preferred_element_type=jnp.float32)

    @pl.when(i == 0)
    def _():
        out_ref[...] = part

    @pl.when(i > 0)
    def _():
        out_ref[...] += part


def _pool(batch2, xf):
    return pl.pallas_call(
        _pool_body,
        grid=(NP // BN,),
        in_specs=[pl.BlockSpec((BN, 1), lambda i: (i, 0)),
                  pl.BlockSpec((BN, D), lambda i: (i, 0))],
        out_specs=pl.BlockSpec((G8, D), lambda i: (0, 0)),
        out_shape=jax.ShapeDtypeStruct((G8, D), jnp.float32),
    )(batch2, xf)


def kernel(x, pos, edge_index, edge_type, batch, Wl, bl, Ws, bs, We, be):
    WeU = We[:, :D, :]
    WeV = We[:, D:2 * D, :]
    WeR = We[:, 2 * D:2 * D + R, :]
    u = We[:, 2 * D + R, :]
    v = We[:, 2 * D + R + 1, :]
    cmat = WeR + be[:, None, :]
    wcat = jnp.concatenate(
        [WeU[i] for i in range(L)] + [WeV[i] for i in range(L)], axis=1)
    uvb = jnp.concatenate(
        [u[:, None, :], v[:, None, :], (bl + bs)[:, None, :],
         jnp.zeros((L, 5, D), jnp.float32)], axis=1)
    wl3 = Wl.reshape(L, R, D, D)

    x_p = jnp.zeros((NP, D), jnp.float32).at[:N].set(x)
    posf = jnp.zeros((3, NP), jnp.float32).at[:, :N].set(pos.T).reshape(-1)
    pad_e = EP - E
    src_p = jnp.concatenate([edge_index[0].astype(jnp.int32),
                             jnp.zeros((pad_e,), jnp.int32)])
    dst_p = jnp.concatenate([edge_index[1].astype(jnp.int32),
                             jnp.full((pad_e,), NP - 1, jnp.int32)])
    rel_p = jnp.concatenate([edge_type[0].astype(jnp.int32),
                             jnp.full((pad_e,), R - 1, jnp.int32)])
    batch_p = jnp.concatenate([batch.astype(jnp.int32),
                               jnp.full((NP - N,), G8, jnp.int32)])

    scl_raw, ilist, counts = _preprocess(posf, src_p, dst_p, rel_p)
    sa = scl_raw[:NSLOT * 8][:R * NP * 8].reshape(R, NP, 8)
    sbp = scl_raw[NSLOT * 8:][:R * NP * 8].reshape(R, NP, 8)

    ab, h = _precompute(x_p, wcat)

    li = x_p
    zeros_nd = jnp.zeros((NP, D), jnp.float32)
    for i in range(L):
        g3 = _scatter(h, ilist, counts)[:R * NP].reshape(R, NP, D)
        an = ab[:, (i + 1) * D:(i + 2) * D] if i < L - 1 else zeros_nd
        bm = ab[:, (L + i) * D:(L + i + 1) * D]
        li, h = _dense_layer(g3, sa, sbp, bm, cmat[i], uvb[i], li, an,
                             wl3[i], Ws[i])

    graph_embedding = _pool(batch_p[:, None], li)
    return li[:N], graph_embedding


# speedup vs baseline: 1.0553x; 1.0553x over previous
import functools

import jax
import jax.numpy as jnp
from jax import lax
from jax.experimental import pallas as pl
from jax.experimental.pallas import tpu as pltpu
from jax.experimental.pallas import tpu_sc as plsc

N = 10000
E = 320000
D = 128
R = 7
L = 4
G8 = 8

NP = 10240
BN = 512
NC = 2
NS = 16
NT = NC * NS
EPT = NP
EP = NT * EPT
CHUNK = 4096
NCH = 18
NSLOT = NCH * CHUNK
CAP = 10240
BLK = 128
ZR = 64
NBLK = NCH * NT * (CAP // BLK)

_mesh = plsc.VectorSubcoreMesh(core_axis_name="c", subcore_axis_name="s")


def _sqrt_sc(sq):
    i = plsc.bitcast(sq, jnp.int32)
    y = plsc.bitcast((i >> 1) + 0x1FBD1DF6, jnp.float32)
    for _ in range(3):
        y = 0.5 * (y + sq / y)
    return y


def _lane(vec, lane):
    m = lax.broadcasted_iota(jnp.int32, (16,), 0) == lane
    return jnp.sum(jnp.where(m, vec, 0))


def _splat(val):
    return jnp.full((16,), val, jnp.int32)


_LANE0 = None


def _pre_body(posf, srcg, dstg, relg,
              scl_out, ilist_out, counts_out,
              pos_v, srcb, dstb, relb,
              seqb, distb, onesb, i0b, i1b, i2b,
              stage_s, stage_l, cntv, zbuf, scl_sh, sem, semw):
    cid = lax.axis_index("c")
    sid = lax.axis_index("s")
    t = cid * NS + sid

    pltpu.sync_copy(posf, pos_v)
    pltpu.sync_copy(srcg.at[pl.ds(t * EPT, EPT)], srcb)
    pltpu.sync_copy(dstg.at[pl.ds(t * EPT, EPT)], dstb)
    pltpu.sync_copy(relg.at[pl.ds(t * EPT, EPT)], relb)

    ones16 = jnp.ones((16,), jnp.float32)
    for kk in range(8):
        onesb[pl.ds(kk * 16, 16)] = ones16

    z16 = jnp.zeros((16,), jnp.float32)
    def zfill(i, _):
        zbuf[pl.ds(i * 16, 16)] = z16
        return 0
    lax.fori_loop(0, 2048 // 16, zfill, 0)
    myz = NSLOT * 8 // NS
    def zloop(i, _):
        pltpu.sync_copy(zbuf, scl_sh.at[pl.ds(sid * myz + i * 2048, 2048)])
        return 0
    lax.fori_loop(0, myz // 2048, zloop, 0)
    plsc.subcore_barrier()

    def p1(j, _):
        base = j * BLK
        for kk in range(BLK // 16):
            o = base + kk * 16
            lo = kk * 16
            s = srcb[pl.ds(o, 16)]
            d = dstb[pl.ds(o, 16)]
            r = relb[pl.ds(o, 16)]
            slot = r * NP + d
            relb[pl.ds(o, 16)] = slot
            e8 = slot * 8
            i0b[pl.ds(lo, 16)] = e8
            i1b[pl.ds(lo, 16)] = e8 + 1
            i2b[pl.ds(lo, 16)] = e8 + 2
            seq = jnp.abs(s - d).astype(jnp.float32)
            px = plsc.load_gather(pos_v, [s])
            py = plsc.load_gather(pos_v, [s + NP])
            pz = plsc.load_gather(pos_v, [s + 2 * NP])
            qx = plsc.load_gather(pos_v, [d])
            qy = plsc.load_gather(pos_v, [d + NP])
            qz = plsc.load_gather(pos_v, [d + 2 * NP])
            dx = px - qx + 1e-6
            dy = py - qy + 1e-6
            dz = pz - qz + 1e-6
            seqb[pl.ds(lo, 16)] = seq
            distb[pl.ds(lo, 16)] = _sqrt_sc(dx * dx + dy * dy + dz * dz)
        pltpu.sync_copy(onesb, scl_sh.at[i0b], add=True)
        pltpu.sync_copy(seqb, scl_sh.at[i1b], add=True)
        pltpu.sync_copy(distb, scl_sh.at[i2b], add=True)
        return 0
    lax.fori_loop(0, EPT // BLK, p1, 0)

    lane0 = lax.broadcasted_iota(jnp.int32, (16,), 0) == 0
    pad_lsl16 = jnp.full((16,), CHUNK, jnp.int32)
    zero16 = jnp.zeros((16,), jnp.int32)
    for c in range(NCH):
        def cstep(i, cur):
            o = i * 16
            sl = relb[pl.ds(o, 16)]
            m = (sl >> 12) == c
            plsc.store_compressed(stage_s.at[pl.ds(cur, 16)],
                                  srcb[pl.ds(o, 16)], mask=m)
            plsc.store_compressed(stage_l.at[pl.ds(cur, 16)],
                                  sl & (CHUNK - 1), mask=m)
            pc = plsc.all_reduce_population_count(m)
            return cur + _lane(pc, 0)
        cursor = lax.fori_loop(0, EPT // 16, cstep, 0)
        target = ((cursor + BLK - 1) // BLK) * BLK
        def padstep(i, _):
            stage_s[pl.ds(cursor + i * 16, 16)] = zero16
            stage_l[pl.ds(cursor + i * 16, 16)] = pad_lsl16
            return 0
        lax.fori_loop(0, (target - cursor + 15) // 16, padstep, 0)
        nb = target // BLK
        xbase = (c * NT + t) * (CAP // BLK)
        def wout(jb, _):
            pltpu.async_copy(stage_s.at[pl.ds(jb * BLK, BLK)],
                             ilist_out.at[xbase + jb, 0], semw)
            pltpu.async_copy(stage_l.at[pl.ds(jb * BLK, BLK)],
                             ilist_out.at[xbase + jb, 1], semw)
            return 0
        lax.fori_loop(0, nb, wout, 0)
        def wdrain(jb, _):
            pltpu.make_async_copy(stage_s.at[pl.ds(0, BLK)],
                                  ilist_out.at[xbase, 0], semw).wait()
            pltpu.make_async_copy(stage_s.at[pl.ds(0, BLK)],
                                  ilist_out.at[xbase, 1], semw).wait()
            return 0
        lax.fori_loop(0, nb, wdrain, 0)
        plsc.store_scatter(cntv, [_splat(c)], jnp.full((16,), nb, jnp.int32),
                           mask=lane0)

    pltpu.sync_copy(cntv, counts_out.at[pl.ds(t * 32, 32)])

    plsc.subcore_barrier()
    pltpu.sync_copy(scl_sh.at[pl.ds(sid * myz, myz)],
                    scl_out.at[pl.ds(cid * (NSLOT * 8) + sid * myz, myz)])


@functools.partial(
    pl.kernel,
    out_type=[
        jax.ShapeDtypeStruct((NC * NSLOT * 8,), jnp.float32),
        jax.ShapeDtypeStruct((NBLK, 2, BLK), jnp.int32),
        jax.ShapeDtypeStruct((NT * 32,), jnp.int32),
    ],
    mesh=_mesh,
    compiler_params=pltpu.CompilerParams(needs_layout_passes=False),
    scratch_types=[
        pltpu.VMEM((3 * NP,), jnp.float32),
        pltpu.VMEM((EPT,), jnp.int32),
        pltpu.VMEM((EPT,), jnp.int32),
        pltpu.VMEM((EPT,), jnp.int32),
        pltpu.VMEM((BLK,), jnp.float32),
        pltpu.VMEM((BLK,), jnp.float32),
        pltpu.VMEM((BLK,), jnp.float32),
        pltpu.VMEM((BLK,), jnp.int32),
        pltpu.VMEM((BLK,), jnp.int32),
        pltpu.VMEM((BLK,), jnp.int32),
        pltpu.VMEM((CAP + 32,), jnp.int32),
        pltpu.VMEM((CAP + 32,), jnp.int32),
        pltpu.VMEM((32,), jnp.int32),
        pltpu.VMEM((2048,), jnp.float32),
        pltpu.VMEM_SHARED((NSLOT * 8,), jnp.float32),
        pltpu.SemaphoreType.DMA,
        pltpu.SemaphoreType.DMA,
    ],
)
def _preprocess(posf, srcg, dstg, relg, scl_out, ilist_out,
                counts_out, *scratch):
    _pre_body(posf, srcg, dstg, relg, scl_out, ilist_out,
              counts_out, *scratch)


def _scat_body(h_hbm, ilist, counts, g_out,
               counts_v, idx2, rowbuf, zv, gbuf,
               si0, si1, si2, sg0, sg1, sg2, ss0, ss1, ss2):
    cid = lax.axis_index("c")
    sid = lax.axis_index("s")
    sems_i = (si0, si1, si2)
    sems_g = (sg0, sg1, sg2)
    sems_s = (ss0, ss1, ss2)

    pltpu.sync_copy(counts, counts_v)
    z16 = jnp.zeros((16,), jnp.float32)
    def zfill(i, _):
        zv[i, pl.ds(0, 16)] = z16
        for kk in range(1, D // 16):
            zv[i, pl.ds(kk * 16, 16)] = z16
        return 0
    lax.fori_loop(0, ZR, zfill, 0)

    rows_per_tile = CHUNK // NS
    for k in range(NCH // NC):
        c = NC * k + cid
        for zz in range(rows_per_tile // ZR):
            pltpu.sync_copy(zv, gbuf.at[pl.ds(sid * rows_per_tile + zz * ZR, ZR)])
        plsc.subcore_barrier()

        t0 = sid
        t1 = sid + NS
        nb0 = _lane(plsc.load_gather(counts_v, [_splat(t0 * 32 + c)]), 0)
        nb1 = _lane(plsc.load_gather(counts_v, [_splat(t1 * 32 + c)]), 0)
        nbt = nb0 + nb1
        x0 = (c * NT + t0) * (CAP // BLK)
        x1 = (c * NT + t1) * (CAP // BLK)

        def start(j, b):
            xb = jnp.where(j < nb0, x0 + j, x1 + (j - nb0))
            pltpu.sync_copy(ilist.at[xb], idx2.at[b])
            pltpu.async_copy(h_hbm.at[idx2.at[b, 0]], rowbuf.at[b],
                             sems_g[b])

        @pl.when(nbt > 0)
        def _():
            start(0, 0)

        def body(j, _):
            for b in range(2):
                @pl.when(j % 2 == b)
                def _():
                    @pl.when(j + 1 < nbt)
                    def _():
                        start(j + 1, 1 - b)
                    pltpu.make_async_copy(h_hbm.at[idx2.at[b, 0]],
                                          rowbuf.at[b], sems_g[b]).wait()
                    pltpu.sync_copy(rowbuf.at[b], gbuf.at[idx2.at[b, 1]],
                                    add=True)
            return 0
        lax.fori_loop(0, nbt, body, 0)

        plsc.subcore_barrier()
        pltpu.sync_copy(
            gbuf.at[pl.ds(sid * rows_per_tile, rows_per_tile)],
            g_out.at[pl.ds(c * CHUNK + sid * rows_per_tile, rows_per_tile)])


@functools.partial(
    pl.kernel,
    out_type=jax.ShapeDtypeStruct((NSLOT, D), jnp.float32),
    mesh=_mesh,
    compiler_params=pltpu.CompilerParams(needs_layout_passes=False),
    scratch_types=[
        pltpu.VMEM((32 * NT,), jnp.int32),
        pltpu.VMEM((3, 2, BLK), jnp.int32),
        pltpu.VMEM((3, BLK, D), jnp.float32),
        pltpu.VMEM((ZR, D), jnp.float32),
        pltpu.VMEM_SHARED((CHUNK + 1, D), jnp.float32),
        pltpu.SemaphoreType.DMA,
        pltpu.SemaphoreType.DMA,
        pltpu.SemaphoreType.DMA,
        pltpu.SemaphoreType.DMA,
        pltpu.SemaphoreType.DMA,
        pltpu.SemaphoreType.DMA,
        pltpu.SemaphoreType.DMA,
        pltpu.SemaphoreType.DMA,
        pltpu.SemaphoreType.DMA,
    ],
)
def _scatter(h_hbm, ilist, counts, g_out, *scratch):
    _scat_body(h_hbm, ilist, counts, g_out, *scratch)


def _dense_layer_body(g_ref, sa_ref, sb_ref, b_ref, c_ref, uv_ref, li_ref,
                      an_ref, wl_ref, ws_ref, h_ref, hn_ref):
    li = li_ref[...]
    acc = jnp.dot(li, ws_ref[...], preferred_element_type=jnp.float32)
    bmat = b_ref[...]
    for r in range(R):
        sa = sa_ref[r]
        sb = sb_ref[r]
        cnt = sa[:, 0:1] + sb[:, 0:1]
        s1 = sa[:, 1:2] + sb[:, 1:2]
        s2 = sa[:, 2:3] + sb[:, 2:3]
        upd = (g_ref[r] + cnt * (bmat + c_ref[r:r + 1, :])
               + s1 * uv_ref[0:1, :] + s2 * uv_ref[1:2, :])
        acc += jnp.dot(upd, wl_ref[r], preferred_element_type=jnp.float32)
    h = jnp.maximum(acc + uv_ref[2:3, :], 0.0) + li
    h_ref[...] = h
    hn_ref[...] = h + an_ref[...]


def _dense_layer(g3, sa, sbp, bmat, cmat, uv, li, an, wl3, ws):
    nb = NP // BN
    return pl.pallas_call(
        _dense_layer_body,
        grid=(nb,),
        in_specs=[
            pl.BlockSpec((R, BN, D), lambda i: (0, i, 0)),
            pl.BlockSpec((R, BN, 8), lambda i: (0, i, 0)),
            pl.BlockSpec((R, BN, 8), lambda i: (0, i, 0)),
            pl.BlockSpec((BN, D), lambda i: (i, 0)),
            pl.BlockSpec((R, D), lambda i: (0, 0)),
            pl.BlockSpec((8, D), lambda i: (0, 0)),
            pl.BlockSpec((BN, D), lambda i: (i, 0)),
            pl.BlockSpec((BN, D), lambda i: (i, 0)),
            pl.BlockSpec((R, D, D), lambda i: (0, 0, 0)),
            pl.BlockSpec((D, D), lambda i: (0, 0)),
        ],
        out_specs=[pl.BlockSpec((BN, D), lambda i: (i, 0)),
                   pl.BlockSpec((BN, D), lambda i: (i, 0))],
        out_shape=[jax.ShapeDtypeStruct((NP, D), jnp.float32)] * 2,
    )(g3, sa, sbp, bmat, cmat, uv, li, an, wl3, ws)


def _precompute_body(x_ref, w_ref, ab_ref, h0_ref):
    xb = x_ref[...]
    ab = jnp.dot(xb, w_ref[...], preferred_element_type=jnp.float32)
    ab_ref[...] = ab
    h0_ref[...] = xb + ab[:, :D]


def _precompute(x, wcat):
    return pl.pallas_call(
        _precompute_body,
        grid=(NP // BN,),
        in_specs=[pl.BlockSpec((BN, D), lambda i: (i, 0)),
                  pl.BlockSpec((D, 2 * L * D), lambda i: (0, 0))],
        out_specs=[pl.BlockSpec((BN, 2 * L * D), lambda i: (i, 0)),
                   pl.BlockSpec((BN, D), lambda i: (i, 0))],
        out_shape=[jax.ShapeDtypeStruct((NP, 2 * L * D), jnp.float32),
                   jax.ShapeDtypeStruct((NP, D), jnp.float32)],
    )(x, wcat)


def _pool_body(b_ref, xf_ref, out_ref):
    i = pl.program_id(0)
    oh = (b_ref[...] == jax.lax.broadcasted_iota(jnp.int32, (1, G8), 1))
    part = jax.lax.dot_general(oh.astype(jnp.float32), xf_ref[...],
                               (((0,), (0,)), ((), ())),
                               preferred_element_type=jnp.float32)

    @pl.when(i == 0)
    def _():
        out_ref[...] = part

    @pl.when(i > 0)
    def _():
        out_ref[...] += part


def _pool(batch2, xf):
    return pl.pallas_call(
        _pool_body,
        grid=(NP // BN,),
        in_specs=[pl.BlockSpec((BN, 1), lambda i: (i, 0)),
                  pl.BlockSpec((BN, D), lambda i: (i, 0))],
        out_specs=pl.BlockSpec((G8, D), lambda i: (0, 0)),
        out_shape=jax.ShapeDtypeStruct((G8, D), jnp.float32),
    )(batch2, xf)


def kernel(x, pos, edge_index, edge_type, batch, Wl, bl, Ws, bs, We, be):
    WeU = We[:, :D, :]
    WeV = We[:, D:2 * D, :]
    WeR = We[:, 2 * D:2 * D + R, :]
    u = We[:, 2 * D + R, :]
    v = We[:, 2 * D + R + 1, :]
    cmat = WeR + be[:, None, :]
    wcat = jnp.concatenate(
        [WeU[i] for i in range(L)] + [WeV[i] for i in range(L)], axis=1)
    uvb = jnp.concatenate(
        [u[:, None, :], v[:, None, :], (bl + bs)[:, None, :],
         jnp.zeros((L, 5, D), jnp.float32)], axis=1)
    wl3 = Wl.reshape(L, R, D, D)

    x_p = jnp.zeros((NP, D), jnp.float32).at[:N].set(x)
    posf = jnp.zeros((3, NP), jnp.float32).at[:, :N].set(pos.T).reshape(-1)
    pad_e = EP - E
    src_p = jnp.concatenate([edge_index[0].astype(jnp.int32),
                             jnp.zeros((pad_e,), jnp.int32)])
    dst_p = jnp.concatenate([edge_index[1].astype(jnp.int32),
                             jnp.full((pad_e,), NP - 1, jnp.int32)])
    rel_p = jnp.concatenate([edge_type[0].astype(jnp.int32),
                             jnp.full((pad_e,), R - 1, jnp.int32)])
    batch_p = jnp.concatenate([batch.astype(jnp.int32),
                               jnp.full((NP - N,), G8, jnp.int32)])

    scl_raw, ilist, counts = _preprocess(posf, src_p, dst_p, rel_p)
    sa = scl_raw[:NSLOT * 8][:R * NP * 8].reshape(R, NP, 8)
    sbp = scl_raw[NSLOT * 8:][:R * NP * 8].reshape(R, NP, 8)

    ab, h = _precompute(x_p, wcat)

    li = x_p
    zeros_nd = jnp.zeros((NP, D), jnp.float32)
    for i in range(L):
        g3 = _scatter(h, ilist, counts)[:R * NP].reshape(R, NP, D)
        an = ab[:, (i + 1) * D:(i + 2) * D] if i < L - 1 else zeros_nd
        bm = ab[:, (L + i) * D:(L + i + 1) * D]
        li, h = _dense_layer(g3, sa, sbp, bm, cmat[i], uvb[i], li, an,
                             wl3[i], Ws[i])

    graph_embedding = _pool(batch_p[:, None], li)
    return li[:N], graph_embedding
